# baseline (speedup 1.0000x reference)
import functools

import jax
import jax.numpy as jnp
from jax import lax
from jax.experimental import pallas as pl
from jax.experimental.pallas import tpu as pltpu
from jax.experimental.pallas import tpu_sc as plsc

N = 10000
E = 320000
F = 128
K = 5
LAMBDA_MAX = 2.0

NC = 2
NS = 16
NW = NC * NS
C = 128
E_PAD = ((E + NW * C - 1) // (NW * C)) * (NW * C)
EPW = E_PAD // NW
NCHUNK = EPW // C
N_PAD = 10240
RPT = N_PAD // NS


def _worker_id():
    return lax.axis_index("s") * NC + lax.axis_index("c")


def _deg_body(rc_hbm, ew_hbm, out_hbm, idx_v, val_v, zb_v, acc_sh):
    c = lax.axis_index("c")
    s = lax.axis_index("s")
    w = _worker_id()

    def zloop(i, _):
        zb_v[pl.ds(i * 16, 16)] = jnp.zeros((16,), jnp.float32)
        return 0
    lax.fori_loop(0, RPT // 16, zloop, 0)
    pltpu.sync_copy(zb_v, acc_sh.at[pl.ds(s * RPT, RPT)])
    plsc.subcore_barrier()

    def chunk(i, _):
        base = w * EPW + i * C
        pltpu.sync_copy(rc_hbm.at[w * NCHUNK + i, 0], idx_v)
        pltpu.sync_copy(ew_hbm.at[pl.ds(base, C)], val_v)
        pltpu.sync_copy(val_v, acc_sh.at[idx_v], add=True)
        return 0
    lax.fori_loop(0, NCHUNK, chunk, 0)
    plsc.subcore_barrier()
    pltpu.sync_copy(acc_sh.at[pl.ds(s * RPT, RPT)], out_hbm.at[c, pl.ds(s * RPT, RPT)])


_deg_call = functools.partial(
    pl.kernel,
    out_type=jax.ShapeDtypeStruct((NC, N_PAD), jnp.float32),
    mesh=plsc.VectorSubcoreMesh(core_axis_name="c", subcore_axis_name="s"),
    compiler_params=pltpu.CompilerParams(needs_layout_passes=False),
    scratch_types=[
        pltpu.VMEM((C,), jnp.int32),
        pltpu.VMEM((C,), jnp.float32),
        pltpu.VMEM((RPT,), jnp.float32),
        pltpu.VMEM_SHARED((N_PAD,), jnp.float32),
    ],
)(_deg_body)


def _lap_body(rc_hbm, ew_hbm, dis_hbm, lap_hbm,
              ridx_v, cidx_v, ew_v, lw_v, dis_v):
    w = _worker_id()
    pltpu.sync_copy(dis_hbm, dis_v)

    def chunk(i, _):
        base = w * EPW + i * C
        pltpu.sync_copy(rc_hbm.at[w * NCHUNK + i, 0], ridx_v)
        pltpu.sync_copy(rc_hbm.at[w * NCHUNK + i, 1], cidx_v)
        pltpu.sync_copy(ew_hbm.at[pl.ds(base, C)], ew_v)
        for j in range(C // 16):
            sl = pl.ds(j * 16, 16)
            dr = plsc.load_gather(dis_v, [ridx_v[sl]])
            dc = plsc.load_gather(dis_v, [cidx_v[sl]])
            lw_v[sl] = (-1.0) * dr * ew_v[sl] * dc
        pltpu.sync_copy(lw_v, lap_hbm.at[pl.ds(base, C)])
        return 0
    lax.fori_loop(0, NCHUNK, chunk, 0)


_lap_call = functools.partial(
    pl.kernel,
    out_type=jax.ShapeDtypeStruct((E_PAD,), jnp.float32),
    mesh=plsc.VectorSubcoreMesh(core_axis_name="c", subcore_axis_name="s"),
    compiler_params=pltpu.CompilerParams(needs_layout_passes=False),
    scratch_types=[
        pltpu.VMEM((C,), jnp.int32),
        pltpu.VMEM((C,), jnp.int32),
        pltpu.VMEM((C,), jnp.float32),
        pltpu.VMEM((C,), jnp.float32),
        pltpu.VMEM((N_PAD,), jnp.float32),
    ],
)(_lap_body)


def _prop_body(h_hbm, rc_hbm, lap_hbm, out_hbm,
               rc_v, lw_v, rows_v, sem_g, sem_s, sem_i, acc_sh):
    c = lax.axis_index("c")
    s = lax.axis_index("s")
    w = _worker_id()

    def zloop(i, _):
        for j in range(F // 16):
            rows_v[0, i, pl.ds(j * 16, 16)] = jnp.zeros((16,), jnp.float32)
        return 0
    lax.fori_loop(0, C, zloop, 0)
    for q in range(RPT // C):
        pltpu.sync_copy(rows_v.at[0], acc_sh.at[pl.ds(s * RPT + q * C, C)])
    plsc.subcore_barrier()

    cbase = w * NCHUNK

    def load_idx(i):
        pltpu.async_copy(rc_hbm.at[cbase + i], rc_v.at[i % 4], sem_i)
        pltpu.async_copy(lap_hbm.at[cbase + i], lw_v.at[i % 4], sem_i)

    def wait_idx():
        pltpu.make_async_copy(rc_hbm.at[0], rc_v.at[0], sem_i).wait()
        pltpu.make_async_copy(lap_hbm.at[0], lw_v.at[0], sem_i).wait()

    def start_gather(i):
        pltpu.async_copy(h_hbm.at[rc_v.at[i % 4, 0]], rows_v.at[i % 2], sem_g)

    def wait_gather():
        pltpu.make_async_copy(h_hbm.at[rc_v.at[0, 0]], rows_v.at[0],
                              sem_g).wait()

    def scale(i):
        b = i % 2
        i4 = i % 4

        def body(g, _):
            lw16 = lw_v[i4, pl.ds(g * 16, 16)]
            for l in range(16):
                e = g * 16 + l
                sv = lw16[l]
                for j in range(F // 16):
                    sl = pl.ds(j * 16, 16)
                    rows_v[b, e, sl] = rows_v[b, e, sl] * sv
            return 0
        lax.fori_loop(0, C // 16, body, 0)

    def start_scatter(i):
        pltpu.async_copy(rows_v.at[i % 2], acc_sh.at[rc_v.at[i % 4, 1]],
                         sem_s, add=True)

    def wait_scatter():
        pltpu.make_async_copy(rows_v.at[0], acc_sh.at[rc_v.at[0, 1]],
                              sem_s).wait()

    load_idx(0)
    wait_idx()
    start_gather(0)
    load_idx(1)
    wait_idx()
    wait_gather()
    start_gather(1)
    load_idx(2)
    scale(0)
    start_scatter(0)

    def steady(i, _):
        wait_idx()
        wait_scatter()
        wait_gather()
        start_gather(i + 1)
        load_idx(i + 2)
        scale(i)
        start_scatter(i)
        return 0
    lax.fori_loop(1, NCHUNK - 2, steady, 0)

    wait_idx()
    wait_scatter()
    wait_gather()
    start_gather(NCHUNK - 1)
    scale(NCHUNK - 2)
    start_scatter(NCHUNK - 2)
    wait_scatter()
    wait_gather()
    scale(NCHUNK - 1)
    start_scatter(NCHUNK - 1)
    wait_scatter()

    plsc.subcore_barrier()
    pltpu.sync_copy(acc_sh.at[pl.ds(s * RPT, RPT)],
                    out_hbm.at[c, pl.ds(s * RPT, RPT)])


_prop_call = functools.partial(
    pl.kernel,
    out_type=jax.ShapeDtypeStruct((NC, N_PAD, F), jnp.float32),
    mesh=plsc.VectorSubcoreMesh(core_axis_name="c", subcore_axis_name="s"),
    compiler_params=pltpu.CompilerParams(needs_layout_passes=False),
    scratch_types=[
        pltpu.VMEM((4, 2, C), jnp.int32),
        pltpu.VMEM((4, C), jnp.float32),
        pltpu.VMEM((2, C, F), jnp.float32),
        pltpu.SemaphoreType.DMA,
        pltpu.SemaphoreType.DMA,
        pltpu.SemaphoreType.DMA,
        pltpu.VMEM_SHARED((N_PAD, F), jnp.float32),
    ],
)(_prop_body)


def _dis_body(deg_ref, out_ref):
    d = deg_ref[0] + deg_ref[1]
    out_ref[...] = jnp.where(d > 0, 1.0 / jnp.sqrt(d), 0.0)


def _dis_call(deg2):
    return pl.pallas_call(
        _dis_body,
        out_shape=jax.ShapeDtypeStruct((N_PAD // 128, 128), jnp.float32),
    )(deg2)


def _combine_body(a, b, p_ref, prev_ref, out_ref):
    out_ref[...] = a * (p_ref[0] + p_ref[1]) - b * prev_ref[...]


def _combine_call(p, prev, a, b):
    blk = 1024
    grid = N_PAD // blk
    return pl.pallas_call(
        functools.partial(_combine_body, a, b),
        grid=(grid,),
        in_specs=[
            pl.BlockSpec((NC, blk, F), lambda i: (0, i, 0)),
            pl.BlockSpec((blk, F), lambda i: (i, 0)),
        ],
        out_specs=pl.BlockSpec((blk, F), lambda i: (i, 0)),
        out_shape=jax.ShapeDtypeStruct((N_PAD, F), jnp.float32),
    )(p, prev)


def _matmul_body(x_ref, w_ref, b_ref, out_ref):
    out_ref[...] = jnp.dot(
        x_ref[...], w_ref[...], preferred_element_type=jnp.float32,
        precision=lax.Precision.HIGHEST) + b_ref[...]


def _matmul_call(xcat, wr, bias):
    blk = 1024
    grid = N_PAD // blk
    return pl.pallas_call(
        _matmul_body,
        grid=(grid,),
        in_specs=[
            pl.BlockSpec((blk, K * F), lambda i: (i, 0)),
            pl.BlockSpec((K * F, F), lambda i: (0, 0)),
            pl.BlockSpec((1, F), lambda i: (0, 0)),
        ],
        out_specs=pl.BlockSpec((blk, F), lambda i: (i, 0)),
        out_shape=jax.ShapeDtypeStruct((N_PAD, F), jnp.float32),
    )(xcat, wr, bias)


def kernel(x, edge_weight, W, bias, edge_index):
    row = jnp.zeros((E_PAD,), jnp.int32).at[:E].set(edge_index[0])
    col = jnp.zeros((E_PAD,), jnp.int32).at[:E].set(edge_index[1])
    ew = jnp.zeros((E_PAD,), jnp.float32).at[:E].set(edge_weight)
    h0 = jnp.zeros((N_PAD, F), jnp.float32).at[:N].set(x)
    ncht = E_PAD // C
    rc = jnp.stack([row.reshape(ncht, C), col.reshape(ncht, C)], axis=1)

    deg2 = _deg_call(rc, ew)
    dis = _dis_call(deg2.reshape(NC, N_PAD // 128, 128)).reshape(N_PAD)
    lap = _lap_call(rc, ew, dis).reshape(ncht, C)

    tx = [h0]
    for k in range(1, K):
        p = _prop_call(tx[-1], rc, lap)
        a, b = (1.0, 0.0) if k == 1 else (2.0, 1.0)
        prev = tx[-1] if k == 1 else tx[-2]
        tx.append(_combine_call(p, prev, a, b))

    xcat = jnp.concatenate(tx, axis=1)
    wr = W.reshape(K * F, F)
    out = _matmul_call(xcat, wr, bias.reshape(1, F))
    return out[:N]
